# Initial kernel scaffold; baseline (speedup 1.0000x reference)
import jax
import jax.numpy as jnp
from jax.experimental import pallas as pl


def kernel(x, edge_index, edge_attr, W0, a_s0, a_d0, b0, W2, a_s2, a_d2, b2):
    raise NotImplementedError("write your pallas kernel here")


# speedup vs baseline: 11.7674x; 11.7674x over previous
import functools

import jax
import jax.numpy as jnp
from jax import lax
from jax.experimental import pallas as pl
from jax.experimental.pallas import tpu as pltpu
from jax.experimental.pallas import tpu_sc as plsc

N = 10000
E = 320000
F_IN = 128
H = 8
HID = 64
C_OUT = 40

NROW = 10240
DUMMY = N
EP = E + N
KB = 128
NTILES = 16
BATCHES = -(-EP // (NTILES * KB))
EPT = BATCHES * KB
EPAD = EPT * NTILES
RPT = NROW // NTILES
TCR = 1024
TCG = NROW // TCR

_f32 = jnp.float32
_EPS = 1e-16


def _dots(x, w_ref, nh):
    return [jnp.dot(x, w_ref[h], preferred_element_type=_f32)
            for h in range(nh)]


def _tc1_body(x_ref, wa_ref, wd_ref, a_ref, d_ref):
    x = x_ref[...]
    for h in range(H):
        a_ref[h] = jnp.dot(x, wa_ref[h], preferred_element_type=_f32)
        d_ref[h] = jnp.dot(x, wd_ref[h], preferred_element_type=_f32)


def _norm_mean(acc_ref, hv):
    tot = None
    for h in range(H):
        m = acc_ref[h, :, 0:hv]
        dn = acc_ref[h, :, hv:hv + 1] + _EPS
        cur = m / dn
        tot = cur if tot is None else tot + cur
    return tot * (1.0 / H)


def _tc2_body(acc_ref, wa_ref, wd_ref, b_ref, a_ref, d_ref):
    x1 = jnp.maximum(_norm_mean(acc_ref, HID) + b_ref[...][0:1, :], 0.0)
    for h in range(H):
        a_ref[h] = jnp.dot(x1, wa_ref[h], preferred_element_type=_f32)
        d_ref[h] = jnp.dot(x1, wd_ref[h], preferred_element_type=_f32)


def _tc3_body(acc_ref, b_ref, o_ref):
    out = _norm_mean(acc_ref, C_OUT) + b_ref[...][0:1, :]
    mx = jnp.max(out, axis=1, keepdims=True)
    lse = mx + jnp.log(jnp.sum(jnp.exp(out - mx), axis=1, keepdims=True))
    o_ref[...] = out - lse


def _tc1(xp, wa, wd):
    ac = HID + 16
    return pl.pallas_call(
        _tc1_body, grid=(TCG,),
        in_specs=[
            pl.BlockSpec((TCR, F_IN), lambda i: (i, 0)),
            pl.BlockSpec((H, F_IN, ac), lambda i: (0, 0, 0)),
            pl.BlockSpec((H, F_IN, 16), lambda i: (0, 0, 0)),
        ],
        out_specs=[
            pl.BlockSpec((H, TCR, ac), lambda i: (0, i, 0)),
            pl.BlockSpec((H, TCR, 16), lambda i: (0, i, 0)),
        ],
        out_shape=[
            jax.ShapeDtypeStruct((H, NROW, ac), _f32),
            jax.ShapeDtypeStruct((H, NROW, 16), _f32),
        ])(xp, wa, wd)


def _tc2(acc, wa, wd, b0t):
    ac_in = HID + 16
    ac = C_OUT + 8
    return pl.pallas_call(
        _tc2_body, grid=(TCG,),
        in_specs=[
            pl.BlockSpec((H, TCR, ac_in), lambda i: (0, i, 0)),
            pl.BlockSpec((H, HID, ac), lambda i: (0, 0, 0)),
            pl.BlockSpec((H, HID, 16), lambda i: (0, 0, 0)),
            pl.BlockSpec((8, HID), lambda i: (0, 0)),
        ],
        out_specs=[
            pl.BlockSpec((H, TCR, ac), lambda i: (0, i, 0)),
            pl.BlockSpec((H, TCR, 16), lambda i: (0, i, 0)),
        ],
        out_shape=[
            jax.ShapeDtypeStruct((H, NROW, ac), _f32),
            jax.ShapeDtypeStruct((H, NROW, 16), _f32),
        ])(acc, wa, wd, b0t)


def _tc3(acc, b2t):
    ac_in = C_OUT + 8
    return pl.pallas_call(
        _tc3_body, grid=(TCG,),
        in_specs=[
            pl.BlockSpec((H, TCR, ac_in), lambda i: (0, i, 0)),
            pl.BlockSpec((8, C_OUT), lambda i: (0, 0)),
        ],
        out_specs=pl.BlockSpec((TCR, C_OUT), lambda i: (i, 0)),
        out_shape=jax.ShapeDtypeStruct((NROW, C_OUT), _f32))(acc, b2t)


def _make_edge_kernel(hv, ac):
    mesh = plsc.VectorSubcoreMesh(core_axis_name="c", subcore_axis_name="s")
    out_type = jax.ShapeDtypeStruct((H, NROW, ac), _f32)
    scratch = [
        pltpu.VMEM((KB,), jnp.int32),
        pltpu.VMEM((KB,), jnp.int32),
        pltpu.VMEM((KB,), jnp.int32),
        pltpu.VMEM((KB, ac), _f32),
        pltpu.VMEM((KB, 16), _f32),
        pltpu.VMEM((KB, ac), _f32),
        pltpu.VMEM((KB,), _f32),
        pltpu.VMEM((64, ac), _f32),
        pltpu.VMEM_SHARED((NROW, ac), _f32),
        pltpu.SemaphoreType.DMA,
        pltpu.SemaphoreType.DMA,
    ]

    @functools.partial(
        pl.kernel, mesh=mesh, out_type=out_type, scratch_types=scratch,
        compiler_params=pltpu.CompilerParams(
            use_tc_tiling_on_sc=False, needs_layout_passes=False))
    def ek(a_t, d_t, srcoff, dstoff, dstp, out,
           src_v, dstg_v, dst_v, abuf, dbuf, mbuf, wbuf, zbuf, acc,
           sem_a, sem_d):
        c = lax.axis_index("c")
        s = lax.axis_index("s")
        io = lax.iota(jnp.int32, 16)
        z16 = jnp.zeros((16,), _f32)
        nv = ac // 16

        def zrow(r, _):
            for v in range(nv):
                zbuf[r, pl.ds(16 * v, 16)] = z16
            return 0
        lax.fori_loop(0, 64, zrow, 0)

        for q in range(4):
            h = 4 * c + q

            def zacc(k, _):
                pltpu.sync_copy(zbuf, acc.at[pl.ds(s * RPT + k * 64, 64)])
                return 0
            lax.fori_loop(0, RPT // 64, zacc, 0)
            plsc.subcore_barrier()

            def batch(b, _):
                base = s * EPT + b * KB
                pltpu.sync_copy(srcoff.at[h, pl.ds(base, KB)], src_v)
                pltpu.sync_copy(dstoff.at[h, pl.ds(base, KB)], dstg_v)
                pltpu.sync_copy(dstp.at[pl.ds(base, KB)], dst_v)
                ca = pltpu.async_copy(a_t.at[src_v], abuf, sem_a)
                cd = pltpu.async_copy(d_t.at[dstg_v], dbuf, sem_d)
                ca.wait()
                cd.wait()
                chv = jnp.full((16,), hv, jnp.int32)
                c0 = jnp.zeros((16,), jnp.int32)
                for j in range(KB // 16):
                    e_vec = j * 16 + io
                    sv = plsc.load_gather(abuf, [e_vec, chv])
                    dv = plsc.load_gather(dbuf, [e_vec, c0])
                    al = sv + dv
                    al = jnp.maximum(al, 0.2 * al)
                    wbuf[pl.ds(j * 16, 16)] = jnp.exp(al)

                def edge(e, _):
                    m0 = plsc.load_gather(
                        wbuf, [jnp.full((16,), e, jnp.int32)])
                    for v in range(hv // 16):
                        lo = v * 16
                        mbuf[e, pl.ds(lo, 16)] = (
                            abuf[e, pl.ds(lo, 16)] * m0)
                    lo = (hv // 16) * 16
                    r = hv - lo
                    if r:
                        sel = jnp.where(io < r, m0, z16)
                        wl = jnp.where(io == r, m0, z16)
                        mbuf[e, pl.ds(lo, 16)] = (
                            abuf[e, pl.ds(lo, 16)] * sel + wl)
                    else:
                        mbuf[e, pl.ds(lo, 16)] = jnp.where(io == 0, m0, z16)
                    return 0
                lax.fori_loop(0, KB, edge, 0)
                pltpu.sync_copy(mbuf, acc.at[dst_v], add=True)
                return 0
            lax.fori_loop(0, BATCHES, batch, 0)

            plsc.subcore_barrier()
            pltpu.sync_copy(acc.at[pl.ds(s * RPT, RPT)],
                            out.at[h, pl.ds(s * RPT, RPT)])
            plsc.subcore_barrier()

    return ek


_edge1 = _make_edge_kernel(HID, HID + 16)
_edge2 = _make_edge_kernel(C_OUT, C_OUT + 8)


def _head_tables(W, a_s, a_d, cin, hv, ac):
    Wr = W.reshape(cin, H, hv)
    ws = jnp.einsum("fhc,hc->fh", Wr, a_s)
    wd = jnp.einsum("fhc,hc->fh", Wr, a_d)
    wa = jnp.concatenate(
        [Wr.transpose(1, 0, 2),
         ws.T[:, :, None],
         jnp.zeros((H, cin, ac - hv - 1), _f32)], axis=2)
    wdt = jnp.concatenate(
        [wd.T[:, :, None], jnp.zeros((H, cin, 15), _f32)], axis=2)
    return wa, wdt


def kernel(x, edge_index, edge_attr, W0, a_s0, a_d0, b0, W2, a_s2, a_d2, b2):
    del edge_attr
    wa0, wd0 = _head_tables(W0, a_s0, a_d0, F_IN, HID, HID + 16)
    wa2, wd2 = _head_tables(W2, a_s2, a_d2, HID, C_OUT, C_OUT + 8)
    b0t = jnp.tile(b0[None, :], (8, 1))
    b2t = jnp.tile(b2[None, :], (8, 1))

    loop = jnp.arange(N, dtype=jnp.int32)
    npad = EPAD - EP
    src = jnp.concatenate([edge_index[0], loop,
                           jnp.zeros((npad,), jnp.int32)])
    dst = jnp.concatenate([edge_index[1], loop,
                           jnp.full((npad,), DUMMY, jnp.int32)])
    hoff = (jnp.arange(H, dtype=jnp.int32) * NROW)[:, None]
    srcoff = src[None, :] + hoff
    dstoff = dst[None, :] + hoff
    xp = jnp.pad(x, ((0, NROW - N), (0, 0)))

    a1, d1 = _tc1(xp, wa0, wd0)
    acc1 = _edge1(a1.reshape(H * NROW, HID + 16),
                  d1.reshape(H * NROW, 16), srcoff, dstoff, dst)
    a2, d2 = _tc2(acc1, wa2, wd2, b0t)
    acc2 = _edge2(a2.reshape(H * NROW, C_OUT + 8),
                  d2.reshape(H * NROW, 16), srcoff, dstoff, dst)
    out = _tc3(acc2, b2t)
    return out[:N]


# speedup vs baseline: 23.2833x; 1.9786x over previous
import functools

import jax
import jax.numpy as jnp
from jax import lax
from jax.experimental import pallas as pl
from jax.experimental.pallas import tpu as pltpu
from jax.experimental.pallas import tpu_sc as plsc

N = 10000
E = 320000
F_IN = 128
H = 8
HID = 64
C_OUT = 40

NROW = 10240
DUMMY = N
EP = E + N
KB = 128
NTILES = 16
BATCHES = -(-EP // (NTILES * KB))
PAIRS = BATCHES // 2
EPT = BATCHES * KB
EPAD = EPT * NTILES
RPT = NROW // NTILES
TCR = 1024
TCG = NROW // TCR

_f32 = jnp.float32
_EPS = 1e-16


def _tc1_body(x_ref, wa_ref, wd_ref, a_ref, d_ref):
    x = x_ref[...]
    for h in range(H):
        a_ref[h] = jnp.dot(x, wa_ref[h], preferred_element_type=_f32)
    d_ref[...] = jnp.dot(x, wd_ref[...], preferred_element_type=_f32)


def _norm_mean(acc_ref, hv):
    tot = None
    for h in range(H):
        m = acc_ref[h, :, 0:hv]
        dn = acc_ref[h, :, hv:hv + 1] + _EPS
        cur = m / dn
        tot = cur if tot is None else tot + cur
    return tot * (1.0 / H)


def _tc2_body(acc_ref, wa_ref, wd_ref, b_ref, a_ref, d_ref):
    x1 = jnp.maximum(_norm_mean(acc_ref, HID) + b_ref[...][0:1, :], 0.0)
    for h in range(H):
        a_ref[h] = jnp.dot(x1, wa_ref[h], preferred_element_type=_f32)
    d_ref[...] = jnp.dot(x1, wd_ref[...], preferred_element_type=_f32)


def _tc3_body(acc_ref, b_ref, o_ref):
    out = _norm_mean(acc_ref, C_OUT) + b_ref[...][0:1, :]
    mx = jnp.max(out, axis=1, keepdims=True)
    lse = mx + jnp.log(jnp.sum(jnp.exp(out - mx), axis=1, keepdims=True))
    o_ref[...] = out - lse


def _tc1(xp, wa, wd):
    ac = HID + 16
    return pl.pallas_call(
        _tc1_body, grid=(TCG,),
        in_specs=[
            pl.BlockSpec((TCR, F_IN), lambda i: (i, 0)),
            pl.BlockSpec((H, F_IN, ac), lambda i: (0, 0, 0)),
            pl.BlockSpec((F_IN, 16), lambda i: (0, 0)),
        ],
        out_specs=[
            pl.BlockSpec((H, TCR, ac), lambda i: (0, i, 0)),
            pl.BlockSpec((TCR, 16), lambda i: (i, 0)),
        ],
        out_shape=[
            jax.ShapeDtypeStruct((H, NROW, ac), _f32),
            jax.ShapeDtypeStruct((NROW, 16), _f32),
        ])(xp, wa, wd)


def _tc2(acc, wa, wd, b0t):
    ac_in = HID + 16
    ac = C_OUT + 8
    return pl.pallas_call(
        _tc2_body, grid=(TCG,),
        in_specs=[
            pl.BlockSpec((H, TCR, ac_in), lambda i: (0, i, 0)),
            pl.BlockSpec((H, HID, ac), lambda i: (0, 0, 0)),
            pl.BlockSpec((HID, 16), lambda i: (0, 0)),
            pl.BlockSpec((8, HID), lambda i: (0, 0)),
        ],
        out_specs=[
            pl.BlockSpec((H, TCR, ac), lambda i: (0, i, 0)),
            pl.BlockSpec((TCR, 16), lambda i: (i, 0)),
        ],
        out_shape=[
            jax.ShapeDtypeStruct((H, NROW, ac), _f32),
            jax.ShapeDtypeStruct((NROW, 16), _f32),
        ])(acc, wa, wd, b0t)


def _tc3(acc, b2t):
    ac_in = C_OUT + 8
    return pl.pallas_call(
        _tc3_body, grid=(TCG,),
        in_specs=[
            pl.BlockSpec((H, TCR, ac_in), lambda i: (0, i, 0)),
            pl.BlockSpec((8, C_OUT), lambda i: (0, 0)),
        ],
        out_specs=pl.BlockSpec((TCR, C_OUT), lambda i: (i, 0)),
        out_shape=jax.ShapeDtypeStruct((NROW, C_OUT), _f32))(acc, b2t)


def _make_edge_kernel(hv, ac):
    mesh = plsc.VectorSubcoreMesh(core_axis_name="c", subcore_axis_name="s")
    out_type = jax.ShapeDtypeStruct((H, NROW, ac), _f32)
    scratch = [
        pltpu.VMEM((EPT,), jnp.int32),
        pltpu.VMEM((KB,), jnp.int32),
        pltpu.VMEM((KB,), jnp.int32),
        pltpu.VMEM((KB,), jnp.int32),
        pltpu.VMEM((KB,), jnp.int32),
        pltpu.VMEM((KB, ac), _f32),
        pltpu.VMEM((KB, ac), _f32),
        pltpu.VMEM((KB, 16), _f32),
        pltpu.VMEM((KB, 16), _f32),
        pltpu.VMEM((KB, ac), _f32),
        pltpu.VMEM((KB, ac), _f32),
        pltpu.VMEM((KB,), jnp.int32),
        pltpu.VMEM((KB,), jnp.int32),
        pltpu.VMEM((KB,), _f32),
        pltpu.VMEM((64, ac), _f32),
        pltpu.VMEM_SHARED((NROW, ac), _f32),
        pltpu.SemaphoreType.DMA,
        pltpu.SemaphoreType.DMA,
        pltpu.SemaphoreType.DMA,
        pltpu.SemaphoreType.DMA,
        pltpu.SemaphoreType.DMA,
        pltpu.SemaphoreType.DMA,
    ]

    @functools.partial(
        pl.kernel, mesh=mesh, out_type=out_type, scratch_types=scratch,
        compiler_params=pltpu.CompilerParams(
            use_tc_tiling_on_sc=False, needs_layout_passes=False))
    def ek(a_t, d_t, pkp, out,
           pk_all, sv0, sv1, dv0, dv1, ab0, ab1, db0, db1,
           mb0, mb1, cv0, cv1, wbuf, zbuf, acc,
           sa0, sa1, sd0, sd1, ss0, ss1):
        c = lax.axis_index("c")
        s = lax.axis_index("s")
        io = lax.iota(jnp.int32, 16)
        z16 = jnp.zeros((16,), _f32)
        svs = (sv0, sv1)
        dvs = (dv0, dv1)
        abs_ = (ab0, ab1)
        dbs = (db0, db1)
        mbs = (mb0, mb1)
        cvs = (cv0, cv1)
        sas = (sa0, sa1)
        sds = (sd0, sd1)
        sss = (ss0, ss1)

        pltpu.sync_copy(pkp.at[pl.ds(s * EPT, EPT)], pk_all)

        def zrow(r, _):
            for v in range(ac // 16):
                zbuf[r, pl.ds(16 * v, 16)] = z16
            return 0
        lax.fori_loop(0, 64, zrow, 0)

        def prep_idx(b, slot, hoffv):
            for k in range(KB // 16):
                o = b * KB + 16 * k
                pk = pk_all[pl.ds(o, 16)]
                svs[slot][pl.ds(16 * k, 16)] = (
                    lax.shift_right_logical(pk, 14) + hoffv)
                dvs[slot][pl.ds(16 * k, 16)] = lax.bitwise_and(pk, 16383)

        def issue_gathers(slot):
            pltpu.async_copy(a_t.at[svs[slot]], abs_[slot], sas[slot])
            pltpu.async_copy(d_t.at[dvs[slot]], dbs[slot], sds[slot])

        def head_body(q, _):
            h = 4 * c + q
            hoffv = jnp.full((16,), 0, jnp.int32) + h * NROW

            def zacc(k, _):
                pltpu.sync_copy(zbuf, acc.at[pl.ds(s * RPT + k * 64, 64)])
                return 0
            lax.fori_loop(0, RPT // 64, zacc, 0)
            plsc.subcore_barrier()

            for slot in range(2):
                prep_idx(jnp.int32(slot), slot, hoffv)
                issue_gathers(slot)

            hvv = jnp.full((16,), hv, jnp.int32)
            hcol = jnp.full((16,), 0, jnp.int32) + h

            def do_batch(b, bp, slot):
                ab, db, mb = abs_[slot], dbs[slot], mbs[slot]
                pltpu.make_async_copy(a_t.at[svs[slot]], ab, sas[slot]).wait()
                pltpu.make_async_copy(d_t.at[dvs[slot]], db, sds[slot]).wait()
                for j in range(KB // 16):
                    e_vec = j * 16 + io
                    sv = plsc.load_gather(ab, [e_vec, hvv])
                    dv = plsc.load_gather(db, [e_vec, hcol])
                    al = sv + dv
                    al = jnp.maximum(al, 0.2 * al)
                    wbuf[pl.ds(j * 16, 16)] = jnp.exp(al)
                @pl.when(bp >= 1)
                def _():
                    pltpu.make_async_copy(mb, acc.at[cvs[slot]],
                                          sss[slot]).wait()
                for k in range(KB // 16):
                    cvs[slot][pl.ds(16 * k, 16)] = dvs[slot][pl.ds(16 * k, 16)]

                def edge2(ee, _):
                    for u in range(2):
                        e = 2 * ee + u
                        m0 = plsc.load_gather(
                            wbuf, [jnp.full((16,), 0, jnp.int32) + e])
                        for v in range(hv // 16):
                            lo = v * 16
                            mb[e, pl.ds(lo, 16)] = ab[e, pl.ds(lo, 16)] * m0
                        lo = (hv // 16) * 16
                        r = hv - lo
                        if r:
                            sel = jnp.where(io < r, m0, z16)
                            wl = jnp.where(io == r, m0, z16)
                            mb[e, pl.ds(lo, 16)] = (
                                ab[e, pl.ds(lo, 16)] * sel + wl)
                        else:
                            mb[e, pl.ds(lo, 16)] = jnp.where(io == 0, m0, z16)
                    return 0
                lax.fori_loop(0, KB // 2, edge2, 0)
                pltpu.async_copy(mb, acc.at[cvs[slot]], sss[slot], add=True)
                @pl.when(b + 2 < BATCHES)
                def _():
                    prep_idx(b + 2, slot, hoffv)
                    issue_gathers(slot)

            def pair(bp, _):
                do_batch(2 * bp, bp, 0)
                do_batch(2 * bp + 1, bp, 1)
                return 0
            lax.fori_loop(0, PAIRS, pair, 0)

            for slot in range(2):
                pltpu.make_async_copy(mbs[slot], acc.at[cvs[slot]],
                                      sss[slot]).wait()
            plsc.subcore_barrier()
            pltpu.sync_copy(acc.at[pl.ds(s * RPT, RPT)],
                            out.at[h, pl.ds(s * RPT, RPT)])
            plsc.subcore_barrier()
            return 0
        lax.fori_loop(0, 4, head_body, 0)

    return ek


_edge1 = _make_edge_kernel(HID, HID + 16)
_edge2 = _make_edge_kernel(C_OUT, C_OUT + 8)


def _head_tables(W, a_s, a_d, cin, hv, ac):
    Wr = W.reshape(cin, H, hv)
    ws = jnp.einsum("fhc,hc->fh", Wr, a_s)
    wd = jnp.einsum("fhc,hc->fh", Wr, a_d)
    wa = jnp.concatenate(
        [Wr.transpose(1, 0, 2),
         ws.T[:, :, None],
         jnp.zeros((H, cin, ac - hv - 1), _f32)], axis=2)
    wdt = jnp.concatenate([wd, jnp.zeros((cin, 16 - H), _f32)], axis=1)
    return wa, wdt


def kernel(x, edge_index, edge_attr, W0, a_s0, a_d0, b0, W2, a_s2, a_d2, b2):
    del edge_attr
    wa0, wd0 = _head_tables(W0, a_s0, a_d0, F_IN, HID, HID + 16)
    wa2, wd2 = _head_tables(W2, a_s2, a_d2, HID, C_OUT, C_OUT + 8)
    b0t = jnp.tile(b0[None, :], (8, 1))
    b2t = jnp.tile(b2[None, :], (8, 1))

    loop = jnp.arange(N, dtype=jnp.int32)
    npad = EPAD - EP
    src = jnp.concatenate([edge_index[0], loop,
                           jnp.zeros((npad,), jnp.int32)])
    dst = jnp.concatenate([edge_index[1], loop,
                           jnp.full((npad,), DUMMY, jnp.int32)])
    xp = jnp.pad(x, ((0, NROW - N), (0, 0)))

    pk = lax.shift_left(src, 14) + dst

    a1, d1 = _tc1(xp, wa0, wd0)
    acc1 = _edge1(a1.reshape(H * NROW, HID + 16), d1, pk)
    a2, d2 = _tc2(acc1, wa2, wd2, b0t)
    acc2 = _edge2(a2.reshape(H * NROW, C_OUT + 8), d2, pk)
    out = _tc3(acc2, b2t)
    return out[:N]


# speedup vs baseline: 48.9538x; 2.1025x over previous
import functools

import jax
import jax.numpy as jnp
from jax import lax
from jax.experimental import pallas as pl
from jax.experimental.pallas import tpu as pltpu
from jax.experimental.pallas import tpu_sc as plsc

N = 10000
E = 320000
F_IN = 128
H = 8
HID = 64
C_OUT = 40

NROW = 10240
DUMMY = N
EP = E + N
KB = 128
NTILES = 16
BATCHES = -(-EP // (NTILES * KB))
PAIRS = BATCHES // 2
EPT = BATCHES * KB
EPAD = EPT * NTILES
RPT = NROW // NTILES
TCR = 1024
TCG = NROW // TCR

_f32 = jnp.float32
_EPS = 1e-16


def _tc1_body(x_ref, wa_ref, wd_ref, a_ref, d_ref):
    x = x_ref[...]
    for h in range(H):
        a_ref[h] = jnp.dot(x, wa_ref[h], preferred_element_type=_f32)
    d_ref[...] = jnp.dot(x, wd_ref[...], preferred_element_type=_f32)


def _norm_mean(acc_ref, hv):
    tot = None
    for h in range(H):
        m = acc_ref[h, :, 0:hv]
        dn = acc_ref[h, :, hv:hv + 1] + _EPS
        cur = m / dn
        tot = cur if tot is None else tot + cur
    return tot * (1.0 / H)


def _tc2_body(acc_ref, wa_ref, wd_ref, b_ref, a_ref, d_ref):
    x1 = jnp.maximum(_norm_mean(acc_ref, HID) + b_ref[...][0:1, :], 0.0)
    for h in range(H):
        a_ref[h] = jnp.dot(x1, wa_ref[h], preferred_element_type=_f32)
    d_ref[...] = jnp.dot(x1, wd_ref[...], preferred_element_type=_f32)


def _tc3_body(acc_ref, b_ref, o_ref):
    out = _norm_mean(acc_ref, C_OUT) + b_ref[...][0:1, :]
    mx = jnp.max(out, axis=1, keepdims=True)
    lse = mx + jnp.log(jnp.sum(jnp.exp(out - mx), axis=1, keepdims=True))
    o_ref[...] = out - lse


def _tc1(xp, wa, wd):
    ac = HID + 16
    return pl.pallas_call(
        _tc1_body, grid=(TCG,),
        in_specs=[
            pl.BlockSpec((TCR, F_IN), lambda i: (i, 0)),
            pl.BlockSpec((H, F_IN, ac), lambda i: (0, 0, 0)),
            pl.BlockSpec((F_IN, 16), lambda i: (0, 0)),
        ],
        out_specs=[
            pl.BlockSpec((H, TCR, ac), lambda i: (0, i, 0)),
            pl.BlockSpec((TCR, 16), lambda i: (i, 0)),
        ],
        out_shape=[
            jax.ShapeDtypeStruct((H, NROW, ac), _f32),
            jax.ShapeDtypeStruct((NROW, 16), _f32),
        ])(xp, wa, wd)


def _tc2(acc, wa, wd, b0t):
    ac_in = HID + 16
    ac = C_OUT + 8
    return pl.pallas_call(
        _tc2_body, grid=(TCG,),
        in_specs=[
            pl.BlockSpec((H, TCR, ac_in), lambda i: (0, i, 0)),
            pl.BlockSpec((H, HID, ac), lambda i: (0, 0, 0)),
            pl.BlockSpec((HID, 16), lambda i: (0, 0)),
            pl.BlockSpec((8, HID), lambda i: (0, 0)),
        ],
        out_specs=[
            pl.BlockSpec((H, TCR, ac), lambda i: (0, i, 0)),
            pl.BlockSpec((TCR, 16), lambda i: (i, 0)),
        ],
        out_shape=[
            jax.ShapeDtypeStruct((H, NROW, ac), _f32),
            jax.ShapeDtypeStruct((NROW, 16), _f32),
        ])(acc, wa, wd, b0t)


def _tc3(acc, b2t):
    ac_in = C_OUT + 8
    return pl.pallas_call(
        _tc3_body, grid=(TCG,),
        in_specs=[
            pl.BlockSpec((H, TCR, ac_in), lambda i: (0, i, 0)),
            pl.BlockSpec((8, C_OUT), lambda i: (0, 0)),
        ],
        out_specs=pl.BlockSpec((TCR, C_OUT), lambda i: (i, 0)),
        out_shape=jax.ShapeDtypeStruct((NROW, C_OUT), _f32))(acc, b2t)


def _make_edge_kernel(hv, ac):
    mesh = plsc.VectorSubcoreMesh(core_axis_name="c", subcore_axis_name="s")
    out_type = jax.ShapeDtypeStruct((H, NROW, ac), _f32)
    scratch = [
        pltpu.VMEM((EPT,), jnp.int32),
        pltpu.VMEM((KB,), jnp.int32),
        pltpu.VMEM((KB,), jnp.int32),
        pltpu.VMEM((KB,), jnp.int32),
        pltpu.VMEM((KB,), jnp.int32),
        pltpu.VMEM((KB, ac), _f32),
        pltpu.VMEM((KB, ac), _f32),
        pltpu.VMEM((KB, 16), _f32),
        pltpu.VMEM((KB, 16), _f32),
        pltpu.VMEM((KB, ac), _f32),
        pltpu.VMEM((KB, ac), _f32),
        pltpu.VMEM((KB,), jnp.int32),
        pltpu.VMEM((KB,), jnp.int32),
        pltpu.VMEM((KB,), _f32),
        pltpu.VMEM((64, ac), _f32),
        pltpu.VMEM_SHARED((NROW, ac), _f32),
        pltpu.SemaphoreType.DMA,
        pltpu.SemaphoreType.DMA,
        pltpu.SemaphoreType.DMA,
        pltpu.SemaphoreType.DMA,
        pltpu.SemaphoreType.DMA,
        pltpu.SemaphoreType.DMA,
    ]

    @functools.partial(
        pl.kernel, mesh=mesh, out_type=out_type, scratch_types=scratch,
        compiler_params=pltpu.CompilerParams(
            use_tc_tiling_on_sc=False, needs_layout_passes=False))
    def ek(a_t, d_t, pkp, out,
           pk_all, sv0, sv1, dv0, dv1, ab0, ab1, db0, db1,
           mb0, mb1, cv0, cv1, wbuf, zbuf, acc,
           sa0, sa1, sd0, sd1, ss0, ss1):
        c = lax.axis_index("c")
        s = lax.axis_index("s")
        io = lax.iota(jnp.int32, 16)
        z16 = jnp.zeros((16,), _f32)
        svs = (sv0, sv1)
        dvs = (dv0, dv1)
        abs_ = (ab0, ab1)
        dbs = (db0, db1)
        mbs = (mb0, mb1)
        cvs = (cv0, cv1)
        sas = (sa0, sa1)
        sds = (sd0, sd1)
        sss = (ss0, ss1)

        pltpu.sync_copy(pkp.at[pl.ds(s * EPT, EPT)], pk_all)

        def zrow(r, _):
            for v in range(ac // 16):
                zbuf[r, pl.ds(16 * v, 16)] = z16
            return 0
        lax.fori_loop(0, 64, zrow, 0)

        def prep_idx(b, slot, hoffv):
            for k in range(KB // 16):
                o = b * KB + 16 * k
                pk = pk_all[pl.ds(o, 16)]
                svs[slot][pl.ds(16 * k, 16)] = (
                    lax.shift_right_logical(pk, 14) + hoffv)
                dvs[slot][pl.ds(16 * k, 16)] = lax.bitwise_and(pk, 16383)

        def issue_gathers(slot):
            pltpu.async_copy(a_t.at[svs[slot]], abs_[slot], sas[slot])
            pltpu.async_copy(d_t.at[dvs[slot]], dbs[slot], sds[slot])

        def head_body(q, _):
            h = 4 * c + q
            hoffv = jnp.full((16,), 0, jnp.int32) + h * NROW

            def zacc(k, _):
                pltpu.sync_copy(zbuf, acc.at[pl.ds(s * RPT + k * 64, 64)])
                return 0
            lax.fori_loop(0, RPT // 64, zacc, 0)
            plsc.subcore_barrier()

            for slot in range(2):
                prep_idx(jnp.int32(slot), slot, hoffv)
                issue_gathers(slot)

            hvv = jnp.full((16,), hv, jnp.int32)
            hcol = jnp.full((16,), 0, jnp.int32) + h

            def do_batch(b, bp, slot):
                ab, db, mb = abs_[slot], dbs[slot], mbs[slot]
                pltpu.make_async_copy(a_t.at[svs[slot]], ab, sas[slot]).wait()
                pltpu.make_async_copy(d_t.at[dvs[slot]], db, sds[slot]).wait()
                for j in range(KB // 16):
                    e_vec = j * 16 + io
                    sv = plsc.load_gather(ab, [e_vec, hvv])
                    dv = plsc.load_gather(db, [e_vec, hcol])
                    al = sv + dv
                    al = jnp.maximum(al, 0.2 * al)
                    wbuf[pl.ds(j * 16, 16)] = jnp.exp(al)
                @pl.when(bp >= 1)
                def _():
                    pltpu.make_async_copy(mb, acc.at[cvs[slot]],
                                          sss[slot]).wait()
                for k in range(KB // 16):
                    cvs[slot][pl.ds(16 * k, 16)] = dvs[slot][pl.ds(16 * k, 16)]

                @functools.partial(plsc.parallel_loop, 0, KB, unroll=4)
                def _edge(e):
                    m0 = plsc.load_gather(
                        wbuf, [jnp.full((16,), 0, jnp.int32) + e])
                    for v in range(hv // 16):
                        lo = v * 16
                        mb[e, pl.ds(lo, 16)] = ab[e, pl.ds(lo, 16)] * m0
                    lo = (hv // 16) * 16
                    r = hv - lo
                    if r:
                        sel = jnp.where(io < r, m0, z16)
                        wl = jnp.where(io == r, m0, z16)
                        mb[e, pl.ds(lo, 16)] = (
                            ab[e, pl.ds(lo, 16)] * sel + wl)
                    else:
                        mb[e, pl.ds(lo, 16)] = jnp.where(io == 0, m0, z16)
                pltpu.async_copy(mb, acc.at[cvs[slot]], sss[slot], add=True)
                @pl.when(b + 2 < BATCHES)
                def _():
                    prep_idx(b + 2, slot, hoffv)
                    issue_gathers(slot)

            def pair(bp, _):
                do_batch(2 * bp, bp, 0)
                do_batch(2 * bp + 1, bp, 1)
                return 0
            lax.fori_loop(0, PAIRS, pair, 0)

            for slot in range(2):
                pltpu.make_async_copy(mbs[slot], acc.at[cvs[slot]],
                                      sss[slot]).wait()
            plsc.subcore_barrier()
            pltpu.sync_copy(acc.at[pl.ds(s * RPT, RPT)],
                            out.at[h, pl.ds(s * RPT, RPT)])
            plsc.subcore_barrier()
            return 0
        lax.fori_loop(0, 4, head_body, 0)

    return ek


_edge1 = _make_edge_kernel(HID, HID + 16)
_edge2 = _make_edge_kernel(C_OUT, C_OUT + 8)


def _head_tables(W, a_s, a_d, cin, hv, ac):
    Wr = W.reshape(cin, H, hv)
    ws = jnp.einsum("fhc,hc->fh", Wr, a_s)
    wd = jnp.einsum("fhc,hc->fh", Wr, a_d)
    wa = jnp.concatenate(
        [Wr.transpose(1, 0, 2),
         ws.T[:, :, None],
         jnp.zeros((H, cin, ac - hv - 1), _f32)], axis=2)
    wdt = jnp.concatenate([wd, jnp.zeros((cin, 16 - H), _f32)], axis=1)
    return wa, wdt


def kernel(x, edge_index, edge_attr, W0, a_s0, a_d0, b0, W2, a_s2, a_d2, b2):
    del edge_attr
    wa0, wd0 = _head_tables(W0, a_s0, a_d0, F_IN, HID, HID + 16)
    wa2, wd2 = _head_tables(W2, a_s2, a_d2, HID, C_OUT, C_OUT + 8)
    b0t = jnp.tile(b0[None, :], (8, 1))
    b2t = jnp.tile(b2[None, :], (8, 1))

    loop = jnp.arange(N, dtype=jnp.int32)
    npad = EPAD - EP
    src = jnp.concatenate([edge_index[0], loop,
                           jnp.zeros((npad,), jnp.int32)])
    dst = jnp.concatenate([edge_index[1], loop,
                           jnp.full((npad,), DUMMY, jnp.int32)])
    xp = jnp.pad(x, ((0, NROW - N), (0, 0)))

    pk = lax.shift_left(src, 14) + dst

    a1, d1 = _tc1(xp, wa0, wd0)
    acc1 = _edge1(a1.reshape(H * NROW, HID + 16), d1, pk)
    a2, d2 = _tc2(acc1, wa2, wd2, b0t)
    acc2 = _edge2(a2.reshape(H * NROW, C_OUT + 8), d2, pk)
    out = _tc3(acc2, b2t)
    return out[:N]
